# baseline (speedup 1.0000x reference)
import functools
import math

import jax
import jax.numpy as jnp
from jax.experimental import pallas as pl
from jax.experimental.pallas import tpu as pltpu

_EPS = 1e-8
_LN_EPS = 1e-5
_NEG_INF = -1e9
_NHEADS = 8


def _gelu_tanh(x):
    c = math.sqrt(2.0 / math.pi)
    return 0.5 * x * (1.0 + jnp.tanh(c * (x + 0.044715 * x * x * x)))


def _ln(y, g, b):
    mu = jnp.mean(y, axis=-1, keepdims=True)
    var = jnp.mean((y - mu) * (y - mu), axis=-1, keepdims=True)
    return (y - mu) * jax.lax.rsqrt(var + _LN_EPS) * g + b


def _fused_kernel(x_ref, *args, bb, s, d, zpm):
    lws = [args[i * 21:(i + 1) * 21] for i in range(2)]
    wz_ref, bz_ref = args[42], args[43]
    zl_ref, zs_ref = args[44], args[45]
    q_scr, k_scr, v_scr, o_scr = args[46:50]

    hd = d // _NHEADS
    dh = d // 2
    R = bb * s

    x = x_ref[...].reshape(R, d)

    rows = jax.lax.broadcasted_iota(jnp.int32, (s, s), 0)
    cols = jax.lax.broadcasted_iota(jnp.int32, (s, s), 1)
    mask = jnp.where(rows >= cols, 0.0, _NEG_INF).astype(jnp.float32)

    for lw in lws:
        (qe_ref, ke_ref, ve_ref, w0_ref, b0_ref, wq_ref, bq_ref, wk_ref,
         bk_ref, wv_ref, bv_ref, wo_ref, bo_ref, g1_ref, be1_ref,
         w1_ref, b1_ref, w2_ref, b2_ref, g2_ref, be2_ref) = lw

        src1 = (jnp.dot(x, w0_ref[...], preferred_element_type=jnp.float32)
                + b0_ref[...])

        def _proj(e_ref, w_ref, b_ref, out_scr):
            w = w_ref[...]
            base = jnp.dot(src1, w[:dh, :], preferred_element_type=jnp.float32)
            emb = (jnp.dot(e_ref[...], w[dh:, :],
                           preferred_element_type=jnp.float32) + b_ref[...])
            out_scr[...] = base.reshape(bb, s, d) + emb[None]

        _proj(qe_ref, wq_ref, bq_ref, q_scr)
        _proj(ke_ref, wk_ref, bk_ref, k_scr)
        _proj(ve_ref, wv_ref, bv_ref, v_scr)

        def _att_body(b, carry):
            qb = q_scr[b]
            kb = k_scr[b]
            vb = v_scr[b]
            parts = []
            for h in range(_NHEADS):
                lo = h * hd
                qh = qb[:, lo:lo + hd]
                kh = kb[:, lo:lo + hd]
                vh = vb[:, lo:lo + hd]
                sc = jax.lax.dot_general(
                    qh, kh, (((1,), (1,)), ((), ())),
                    preferred_element_type=jnp.float32) + mask
                m = jnp.max(sc, axis=-1, keepdims=True)
                p = jnp.exp(sc - m)
                p = p / jnp.sum(p, axis=-1, keepdims=True)
                parts.append(jnp.dot(p, vh,
                                     preferred_element_type=jnp.float32))
            o_scr[b] = jnp.concatenate(parts, axis=1)
            return carry

        jax.lax.fori_loop(0, bb, _att_body, 0)

        attn = (jnp.dot(o_scr[...].reshape(R, d), wo_ref[...],
                        preferred_element_type=jnp.float32) + bo_ref[...])
        x1 = _ln(x + attn, g1_ref[...], be1_ref[...])
        h1 = _gelu_tanh(jnp.dot(x1, w1_ref[...],
                                preferred_element_type=jnp.float32)
                        + b1_ref[...])
        h2 = (jnp.dot(h1, w2_ref[...], preferred_element_type=jnp.float32)
              + b2_ref[...])
        x = _ln(x1 + h2, g2_ref[...], be2_ref[...])

    z = jnp.dot(x, wz_ref[...], preferred_element_type=jnp.float32) + bz_ref[...]
    zl_ref[...] = (z[:, :zpm] + _EPS).reshape(bb, s, zpm)
    zs_ref[...] = (jnp.log(1.0 + jnp.exp(z[:, zpm:])) + _EPS).reshape(bb, s, zpm)


def _full_spec(a):
    return pl.BlockSpec(a.shape, lambda i, n=a.ndim: (0,) * n)


def kernel(t1, t2, pe, heads_wlt, heads_bl, heads_wst, heads_bs,
           l0_qe, l0_ke, l0_ve, l0_w0t, l0_b0, l0_wqt, l0_bq, l0_wkt, l0_bk,
           l0_wvt, l0_bv, l0_wot, l0_bo, l0_g1, l0_be1, l0_w1t, l0_b1,
           l0_w2t, l0_b2, l0_g2, l0_be2,
           l1_qe, l1_ke, l1_ve, l1_w0t, l1_b0, l1_wqt, l1_bq, l1_wkt, l1_bk,
           l1_wvt, l1_bv, l1_wot, l1_bo, l1_g1, l1_be1, l1_w1t, l1_b1,
           l1_w2t, l1_b2, l1_g2, l1_be2):
    B, seq_len, _ = t1.shape
    S, d = pe.shape
    zpm = heads_wlt.shape[1]
    hd = d // _NHEADS
    scale = 1.0 / math.sqrt(hd)

    h1 = t1[:, 0, :].reshape(B, -1, d)
    h2 = t2[:, 0, :].reshape(B, -1, d)
    h0 = jnp.concatenate([h1, h2], axis=1) + pe[None]

    l0_wqs, l0_bqs = l0_wqt * scale, l0_bq * scale
    l1_wqs, l1_bqs = l1_wqt * scale, l1_bq * scale

    wz = jnp.concatenate([heads_wlt, heads_wst], axis=1)
    bz = jnp.concatenate([heads_bl, heads_bs], axis=1)

    if B % 16 == 0:
        bb = 16
    elif B % 4 == 0:
        bb = 4
    else:
        bb = 1

    weights = (l0_qe, l0_ke, l0_ve, l0_w0t, l0_b0, l0_wqs, l0_bqs,
               l0_wkt, l0_bk, l0_wvt, l0_bv, l0_wot, l0_bo,
               l0_g1, l0_be1, l0_w1t, l0_b1, l0_w2t, l0_b2, l0_g2, l0_be2,
               l1_qe, l1_ke, l1_ve, l1_w0t, l1_b0, l1_wqs, l1_bqs,
               l1_wkt, l1_bk, l1_wvt, l1_bv, l1_wot, l1_bo,
               l1_g1, l1_be1, l1_w1t, l1_b1, l1_w2t, l1_b2, l1_g2, l1_be2,
               wz, bz)

    in_specs = ([pl.BlockSpec((bb, S, d), lambda i: (i, 0, 0))]
                + [_full_spec(w) for w in weights])
    out_specs = (pl.BlockSpec((bb, S, zpm), lambda i: (i, 0, 0)),
                 pl.BlockSpec((bb, S, zpm), lambda i: (i, 0, 0)))

    per_b = (2 * S * d * (d // 2) + 3 * 4 * S * (d // 2) * d
             + 4 * S * S * d + 2 * S * d * d + 4 * S * d * d
             + 4 * S * d * zpm)
    flops = int(2 * B * per_b)
    transcendentals = int(2 * B * (_NHEADS * S * S + 2 * S * d + 4 * S))
    bytes_accessed = int(4 * (2 * B * S * d
                              + sum(int(w.size) for w in weights)))

    body = functools.partial(_fused_kernel, bb=bb, s=S, d=d, zpm=zpm)
    zl, zs = pl.pallas_call(
        body,
        out_shape=(jax.ShapeDtypeStruct((B, S, zpm), jnp.float32),
                   jax.ShapeDtypeStruct((B, S, zpm), jnp.float32)),
        grid=(B // bb,),
        in_specs=in_specs,
        out_specs=out_specs,
        scratch_shapes=[pltpu.VMEM((bb, S, d), jnp.float32)
                        for _ in range(4)],
        compiler_params=pltpu.CompilerParams(
            dimension_semantics=("parallel",)),
        cost_estimate=pl.CostEstimate(flops=flops,
                                      transcendentals=transcendentals,
                                      bytes_accessed=bytes_accessed),
    )(h0, *weights)

    loc = jnp.broadcast_to(zl.reshape(B, 1, S * zpm), (B, seq_len, S * zpm))
    scl = jnp.broadcast_to(zs.reshape(B, 1, S * zpm), (B, seq_len, S * zpm))
    return {"loc": loc, "scale": scl}


# speedup vs baseline: 3.6556x; 3.6556x over previous
import functools
import math

import jax
import jax.numpy as jnp
from jax.experimental import pallas as pl
from jax.experimental.pallas import tpu as pltpu

_EPS = 1e-8
_LN_EPS = 1e-5
_NEG_INF = -1e9
_NHEADS = 8


def _gelu_tanh(x):
    c = math.sqrt(2.0 / math.pi)
    return 0.5 * x * (1.0 + jnp.tanh(c * (x + 0.044715 * x * x * x)))


def _ln(y, g, b):
    mu = jnp.mean(y, axis=-1, keepdims=True)
    var = jnp.mean((y - mu) * (y - mu), axis=-1, keepdims=True)
    return (y - mu) * jax.lax.rsqrt(var + _LN_EPS) * g + b


def _fused_kernel(x_ref, *args, bb, s, d, zpm):
    lws = [args[i * 21:(i + 1) * 21] for i in range(2)]
    wz_ref, bz_ref = args[42], args[43]
    zl_ref, zs_ref = args[44], args[45]

    hd = d // _NHEADS
    dh = d // 2
    R = bb * s

    x = x_ref[...].reshape(R, d)

    rows = jax.lax.broadcasted_iota(jnp.int32, (s, s), 0)
    cols = jax.lax.broadcasted_iota(jnp.int32, (s, s), 1)
    mask = jnp.where(rows >= cols, 0.0, _NEG_INF).astype(jnp.float32)

    for lw in lws:
        (qe_ref, ke_ref, ve_ref, w0_ref, b0_ref, wq_ref, bq_ref, wk_ref,
         bk_ref, wv_ref, bv_ref, wo_ref, bo_ref, g1_ref, be1_ref,
         w1_ref, b1_ref, w2_ref, b2_ref, g2_ref, be2_ref) = lw

        src1 = (jnp.dot(x, w0_ref[...], preferred_element_type=jnp.float32)
                + b0_ref[...])

        def _proj(e_ref, w_ref, b_ref):
            w = w_ref[...]
            base = jnp.dot(src1, w[:dh, :], preferred_element_type=jnp.float32)
            emb = (jnp.dot(e_ref[...], w[dh:, :],
                           preferred_element_type=jnp.float32) + b_ref[...])
            return base.reshape(bb, s, d) + emb[None]

        q3 = _proj(qe_ref, wq_ref, bq_ref)
        k3 = _proj(ke_ref, wk_ref, bk_ref)
        v3 = _proj(ve_ref, wv_ref, bv_ref)

        parts = []
        for h in range(_NHEADS):
            lo = h * hd
            qh = q3[:, :, lo:lo + hd]
            kh = k3[:, :, lo:lo + hd]
            vh = v3[:, :, lo:lo + hd]
            sc = jax.lax.dot_general(
                qh, kh, (((2,), (2,)), ((0,), (0,))),
                preferred_element_type=jnp.float32) + mask[None]
            m = jnp.max(sc, axis=-1, keepdims=True)
            p = jnp.exp(sc - m)
            p = p / jnp.sum(p, axis=-1, keepdims=True)
            parts.append(jax.lax.dot_general(
                p, vh, (((2,), (1,)), ((0,), (0,))),
                preferred_element_type=jnp.float32))
        o3 = jnp.concatenate(parts, axis=-1)

        attn = (jnp.dot(o3.reshape(R, d), wo_ref[...],
                        preferred_element_type=jnp.float32) + bo_ref[...])
        x1 = _ln(x + attn, g1_ref[...], be1_ref[...])
        h1 = _gelu_tanh(jnp.dot(x1, w1_ref[...],
                                preferred_element_type=jnp.float32)
                        + b1_ref[...])
        h2 = (jnp.dot(h1, w2_ref[...], preferred_element_type=jnp.float32)
              + b2_ref[...])
        x = _ln(x1 + h2, g2_ref[...], be2_ref[...])

    z = jnp.dot(x, wz_ref[...], preferred_element_type=jnp.float32) + bz_ref[...]
    zl_ref[...] = (z[:, :zpm] + _EPS).reshape(bb, s, zpm)
    zs_ref[...] = (jnp.log(1.0 + jnp.exp(z[:, zpm:])) + _EPS).reshape(bb, s, zpm)


def _full_spec(a):
    return pl.BlockSpec(a.shape, lambda i, n=a.ndim: (0,) * n)


def kernel(t1, t2, pe, heads_wlt, heads_bl, heads_wst, heads_bs,
           l0_qe, l0_ke, l0_ve, l0_w0t, l0_b0, l0_wqt, l0_bq, l0_wkt, l0_bk,
           l0_wvt, l0_bv, l0_wot, l0_bo, l0_g1, l0_be1, l0_w1t, l0_b1,
           l0_w2t, l0_b2, l0_g2, l0_be2,
           l1_qe, l1_ke, l1_ve, l1_w0t, l1_b0, l1_wqt, l1_bq, l1_wkt, l1_bk,
           l1_wvt, l1_bv, l1_wot, l1_bo, l1_g1, l1_be1, l1_w1t, l1_b1,
           l1_w2t, l1_b2, l1_g2, l1_be2):
    B, seq_len, _ = t1.shape
    S, d = pe.shape
    zpm = heads_wlt.shape[1]
    hd = d // _NHEADS
    scale = 1.0 / math.sqrt(hd)

    h1 = t1[:, 0, :].reshape(B, -1, d)
    h2 = t2[:, 0, :].reshape(B, -1, d)
    h0 = jnp.concatenate([h1, h2], axis=1) + pe[None]

    l0_wqs, l0_bqs = l0_wqt * scale, l0_bq * scale
    l1_wqs, l1_bqs = l1_wqt * scale, l1_bq * scale

    wz = jnp.concatenate([heads_wlt, heads_wst], axis=1)
    bz = jnp.concatenate([heads_bl, heads_bs], axis=1)

    if B % 16 == 0:
        bb = 16
    elif B % 4 == 0:
        bb = 4
    else:
        bb = 1

    weights = (l0_qe, l0_ke, l0_ve, l0_w0t, l0_b0, l0_wqs, l0_bqs,
               l0_wkt, l0_bk, l0_wvt, l0_bv, l0_wot, l0_bo,
               l0_g1, l0_be1, l0_w1t, l0_b1, l0_w2t, l0_b2, l0_g2, l0_be2,
               l1_qe, l1_ke, l1_ve, l1_w0t, l1_b0, l1_wqs, l1_bqs,
               l1_wkt, l1_bk, l1_wvt, l1_bv, l1_wot, l1_bo,
               l1_g1, l1_be1, l1_w1t, l1_b1, l1_w2t, l1_b2, l1_g2, l1_be2,
               wz, bz)

    in_specs = ([pl.BlockSpec((bb, S, d), lambda i: (i, 0, 0))]
                + [_full_spec(w) for w in weights])
    out_specs = (pl.BlockSpec((bb, S, zpm), lambda i: (i, 0, 0)),
                 pl.BlockSpec((bb, S, zpm), lambda i: (i, 0, 0)))

    per_b = (2 * S * d * (d // 2) + 3 * 4 * S * (d // 2) * d
             + 4 * S * S * d + 2 * S * d * d + 4 * S * d * d
             + 4 * S * d * zpm)
    flops = int(2 * B * per_b)
    transcendentals = int(2 * B * (_NHEADS * S * S + 2 * S * d + 4 * S))
    bytes_accessed = int(4 * (2 * B * S * d
                              + sum(int(w.size) for w in weights)))

    body = functools.partial(_fused_kernel, bb=bb, s=S, d=d, zpm=zpm)
    zl, zs = pl.pallas_call(
        body,
        out_shape=(jax.ShapeDtypeStruct((B, S, zpm), jnp.float32),
                   jax.ShapeDtypeStruct((B, S, zpm), jnp.float32)),
        grid=(B // bb,),
        in_specs=in_specs,
        out_specs=out_specs,
        compiler_params=pltpu.CompilerParams(
            dimension_semantics=("parallel",)),
        cost_estimate=pl.CostEstimate(flops=flops,
                                      transcendentals=transcendentals,
                                      bytes_accessed=bytes_accessed),
    )(h0, *weights)

    loc = jnp.broadcast_to(zl.reshape(B, 1, S * zpm), (B, seq_len, S * zpm))
    scl = jnp.broadcast_to(zs.reshape(B, 1, S * zpm), (B, seq_len, S * zpm))
    return {"loc": loc, "scale": scl}
